# baseline (speedup 1.0000x reference)
import functools

import jax
import jax.numpy as jnp
from jax import lax
from jax.experimental import pallas as pl
from jax.experimental.pallas import tpu as pltpu
from jax.experimental.pallas import tpu_sc as plsc

N = 10000
NPAD = 10240
NC = 2
NS = 16
ROWS_PER_TILE = NPAD // NS
E = 160000
K = 128
NCH0 = 62
NCH1 = 18
EPAD = NS * (NCH0 + NCH1) * K
BM = 512

_MESH = plsc.VectorSubcoreMesh(core_axis_name="c", subcore_axis_name="s")


def _zero_fill(buf):
    def fill(i, carry):
        buf[i // 8, pl.ds((i % 8) * 16, 16)] = jnp.zeros((16,), jnp.float32)
        return carry

    lax.fori_loop(0, K * 8, fill, 0)


def _make_edge_pass(d):
    @functools.partial(
        pl.kernel,
        out_type=jax.ShapeDtypeStruct((NC, NPAD, d), jnp.float32),
        mesh=_MESH,
        scratch_types=[
            pltpu.VMEM((NCH0, K), jnp.int32),
            pltpu.VMEM((NCH0, K), jnp.int32),
            pltpu.VMEM((2, K, d), jnp.float32),
            pltpu.VMEM_SHARED((NPAD, d), jnp.float32),
            pltpu.SemaphoreType.DMA,
            pltpu.SemaphoreType.DMA,
            pltpu.SemaphoreType.DMA,
            pltpu.SemaphoreType.DMA,
        ],
    )
    def edge_pass(h_hbm, src0_hbm, dst0_hbm, src1_hbm, dst1_hbm, out_hbm,
                  src_v, dst_v, rows_v, acc, gs0, gs1, ss0, ss1):
        c = lax.axis_index("c")
        s = lax.axis_index("s")
        r0 = s * ROWS_PER_TILE
        rows = pl.ds(r0, ROWS_PER_TILE)
        ncw = jnp.where(c == 0, NCH0, NCH1)
        gsem = (gs0, gs1)
        ssem = (ss0, ss1)

        def g_start(i, b):
            pltpu.async_copy(h_hbm.at[src_v.at[i]], rows_v.at[b], gsem[b])

        def g_wait(b):
            pltpu.make_async_copy(h_hbm.at[src_v.at[0]], rows_v.at[b], gsem[b]).wait()

        def s_start(i, b):
            pltpu.async_copy(rows_v.at[b], acc.at[dst_v.at[i]], ssem[b], add=True)

        def s_wait(b):
            pltpu.make_async_copy(rows_v.at[b], acc.at[dst_v.at[0]], ssem[b]).wait()

        @pl.when(c == 0)
        def _():
            pltpu.sync_copy(src0_hbm.at[s], src_v.at[pl.ds(0, NCH0)])
            pltpu.sync_copy(dst0_hbm.at[s], dst_v.at[pl.ds(0, NCH0)])

        @pl.when(c != 0)
        def _():
            pltpu.sync_copy(src1_hbm.at[s], src_v.at[pl.ds(0, NCH1)])
            pltpu.sync_copy(dst1_hbm.at[s], dst_v.at[pl.ds(0, NCH1)])

        _zero_fill(rows_v.at[0])
        for j in range(ROWS_PER_TILE // K):
            pltpu.sync_copy(rows_v.at[0], acc.at[pl.ds(r0 + j * K, K)])
        plsc.subcore_barrier()

        g_start(0, 0)

        def pair(j, carry):
            for b in (0, 1):
                i = 2 * j + b

                @pl.when(i >= 1)
                def _():
                    s_wait(1 - b)

                @pl.when(i + 1 < ncw)
                def _():
                    g_start(i + 1, 1 - b)

                g_wait(b)
                s_start(i, b)
            return carry

        lax.fori_loop(0, ncw // 2, pair, 0)
        s_wait(1)
        plsc.subcore_barrier()
        pltpu.sync_copy(acc.at[rows], out_hbm.at[c, rows])

    return edge_pass


_edge_pass_128 = _make_edge_pass(128)


@functools.partial(
    pl.kernel,
    out_type=jax.ShapeDtypeStruct((NC, NPAD, 128), jnp.float32),
    mesh=_MESH,
    scratch_types=[
        pltpu.VMEM((NCH0, K), jnp.int32),
        pltpu.VMEM((K, 128), jnp.float32),
        pltpu.VMEM_SHARED((NPAD, 128), jnp.float32),
        pltpu.SemaphoreType.DMA,
    ],
)
def _deg_pass(dst0_hbm, dst1_hbm, out_hbm, dst_v, ones_v, acc, sem):
    c = lax.axis_index("c")
    s = lax.axis_index("s")
    r0 = s * ROWS_PER_TILE
    rows = pl.ds(r0, ROWS_PER_TILE)
    ncw = jnp.where(c == 0, NCH0, NCH1)

    @pl.when(c == 0)
    def _():
        pltpu.sync_copy(dst0_hbm.at[s], dst_v.at[pl.ds(0, NCH0)])

    @pl.when(c != 0)
    def _():
        pltpu.sync_copy(dst1_hbm.at[s], dst_v.at[pl.ds(0, NCH1)])

    _zero_fill(ones_v)
    for j in range(ROWS_PER_TILE // K):
        pltpu.sync_copy(ones_v, acc.at[pl.ds(r0 + j * K, K)])

    def fill1(i, carry):
        ones_v[i // 8, pl.ds((i % 8) * 16, 16)] = jnp.ones((16,), jnp.float32)
        return carry

    lax.fori_loop(0, K * 8, fill1, 0)
    plsc.subcore_barrier()

    def chunk(i, carry):
        pltpu.async_copy(ones_v, acc.at[dst_v.at[i]], sem, add=True)
        return carry

    lax.fori_loop(0, ncw, chunk, 0)

    def drain(i, carry):
        pltpu.make_async_copy(ones_v, acc.at[dst_v.at[0]], sem).wait()
        return carry

    lax.fori_loop(0, ncw, drain, 0)
    plsc.subcore_barrier()
    pltpu.sync_copy(acc.at[rows], out_hbm.at[c, rows])


def _mm1(xp, w1, p0, p1):
    def body(x_ref, w_ref, p0_ref, p1_ref, h_ref, dinv_ref):
        deg = 1.0 + p0_ref[:, 0:1] + p1_ref[:, 0:1]
        dv = lax.rsqrt(deg)
        h = jnp.dot(x_ref[...], w_ref[...], preferred_element_type=jnp.float32)
        h_ref[...] = h * dv
        dinv_ref[...] = dv

    return pl.pallas_call(
        body,
        grid=(NPAD // BM,),
        in_specs=[
            pl.BlockSpec((BM, 384), lambda i: (i, 0)),
            pl.BlockSpec((384, 128), lambda i: (0, 0)),
            pl.BlockSpec((BM, 128), lambda i: (i, 0)),
            pl.BlockSpec((BM, 128), lambda i: (i, 0)),
        ],
        out_specs=[
            pl.BlockSpec((BM, 128), lambda i: (i, 0)),
            pl.BlockSpec((BM, 1), lambda i: (i, 0)),
        ],
        out_shape=[
            jax.ShapeDtypeStruct((NPAD, 128), jnp.float32),
            jax.ShapeDtypeStruct((NPAD, 1), jnp.float32),
        ],
    )(xp, w1, p0, p1)


def _epi_mm(p0, p1, h, dinv, b, w):
    def body(p0_ref, p1_ref, h_ref, dinv_ref, b_ref, w_ref, o_ref):
        dv = dinv_ref[...]
        act = jnp.maximum(
            dv * (p0_ref[...] + p1_ref[...] + h_ref[...]) + b_ref[...], 0.0)
        o_ref[...] = dv * jnp.dot(act, w_ref[...], preferred_element_type=jnp.float32)

    return pl.pallas_call(
        body,
        grid=(NPAD // BM,),
        in_specs=[
            pl.BlockSpec((BM, 128), lambda i: (i, 0)),
            pl.BlockSpec((BM, 128), lambda i: (i, 0)),
            pl.BlockSpec((BM, 128), lambda i: (i, 0)),
            pl.BlockSpec((BM, 1), lambda i: (i, 0)),
            pl.BlockSpec((1, 128), lambda i: (0, 0)),
            pl.BlockSpec((128, 128), lambda i: (0, 0)),
        ],
        out_specs=pl.BlockSpec((BM, 128), lambda i: (i, 0)),
        out_shape=jax.ShapeDtypeStruct((NPAD, 128), jnp.float32),
    )(p0, p1, h, dinv, b, w)


def _epi_final(p0, p1, h, dinv, b):
    def body(p0_ref, p1_ref, h_ref, dinv_ref, b_ref, o_ref):
        o_ref[...] = (dinv_ref[...] * (p0_ref[...] + p1_ref[...] + h_ref[...])
                      + b_ref[...])

    return pl.pallas_call(
        body,
        grid=(NPAD // BM,),
        in_specs=[
            pl.BlockSpec((BM, 128), lambda i: (i, 0)),
            pl.BlockSpec((BM, 128), lambda i: (i, 0)),
            pl.BlockSpec((BM, 128), lambda i: (i, 0)),
            pl.BlockSpec((BM, 1), lambda i: (i, 0)),
            pl.BlockSpec((1, 128), lambda i: (0, 0)),
        ],
        out_specs=pl.BlockSpec((BM, 128), lambda i: (i, 0)),
        out_shape=jax.ShapeDtypeStruct((NPAD, 128), jnp.float32),
    )(p0, p1, h, dinv, b)


def kernel(x, edge_index, W1, b1, W2, b2, W3, b3, W4, b4, W5, b5, W6, b6):
    xp = jnp.pad(x, ((0, NPAD - N), (0, 0)))
    srcf = jnp.concatenate([edge_index[0], jnp.zeros((EPAD - E,), jnp.int32)])
    pad_dst = N + jnp.arange(EPAD - E, dtype=jnp.int32) % (NPAD - N)
    dstf = jnp.concatenate([edge_index[1], pad_dst])
    cut = NS * NCH0 * K
    src0 = srcf[:cut].reshape(NS, NCH0, K)
    src1 = srcf[cut:].reshape(NS, NCH1, K)
    dst0 = dstf[:cut].reshape(NS, NCH0, K)
    dst1 = dstf[cut:].reshape(NS, NCH1, K)
    w1p = jnp.pad(W1, ((0, 0), (0, 16)))
    wp = [jnp.pad(w, ((0, 16), (0, 128 - w.shape[1]))) for w in (W2, W3, W4, W5, W6)]
    bp = [jnp.pad(b, (0, 128 - b.shape[0])).reshape(1, 128) for b in (b1, b2, b3, b4, b5, b6)]

    degp = _deg_pass(dst0, dst1)
    h, dinv = _mm1(xp, w1p, degp[0], degp[1])

    for bb, ww in zip(bp[:5], wp):
        p = _edge_pass_128(h, src0, dst0, src1, dst1)
        h = _epi_mm(p[0], p[1], h, dinv, bb, ww)

    p = _edge_pass_128(h, src0, dst0, src1, dst1)
    out = _epi_final(p[0], p[1], h, dinv, bp[5])
    return out[:N, :2]


# speedup vs baseline: 1.0853x; 1.0853x over previous
import functools

import jax
import jax.numpy as jnp
from jax import lax
from jax.experimental import pallas as pl
from jax.experimental.pallas import tpu as pltpu
from jax.experimental.pallas import tpu_sc as plsc

N = 10000
NPAD = 10240
NC = 2
NS = 16
ROWS_PER_TILE = NPAD // NS
E = 160000
K = 128
NCH0 = 60
NCH1 = 20
EPAD = NS * (NCH0 + NCH1) * K
BM = 512

_MESH = plsc.VectorSubcoreMesh(core_axis_name="c", subcore_axis_name="s")


def _zero_fill(buf):
    def fill(i, carry):
        buf[i // 8, pl.ds((i % 8) * 16, 16)] = jnp.zeros((16,), jnp.float32)
        return carry

    lax.fori_loop(0, K * 8, fill, 0)


def _make_edge_pass(d):
    @functools.partial(
        pl.kernel,
        out_type=jax.ShapeDtypeStruct((NC, NPAD, d), jnp.float32),
        mesh=_MESH,
        scratch_types=[
            pltpu.VMEM((NCH0, K), jnp.int32),
            pltpu.VMEM((NCH0, K), jnp.int32),
            pltpu.VMEM((2, K, d), jnp.float32),
            pltpu.VMEM_SHARED((NPAD, d), jnp.float32),
            pltpu.SemaphoreType.DMA,
            pltpu.SemaphoreType.DMA,
            pltpu.SemaphoreType.DMA,
            pltpu.SemaphoreType.DMA,
        ],
    )
    def edge_pass(h_hbm, src0_hbm, dst0_hbm, src1_hbm, dst1_hbm, out_hbm,
                  src_v, dst_v, rows_v, acc, gs0, gs1, ss0, ss1):
        c = lax.axis_index("c")
        s = lax.axis_index("s")
        r0 = s * ROWS_PER_TILE
        rows = pl.ds(r0, ROWS_PER_TILE)
        ncw = jnp.where(c == 0, NCH0, NCH1)
        gsem = (gs0, gs1)
        ssem = (ss0, ss1)

        def g_start(i, b):
            pltpu.async_copy(h_hbm.at[src_v.at[i]], rows_v.at[b], gsem[b])

        def g_wait(b):
            pltpu.make_async_copy(h_hbm.at[src_v.at[0]], rows_v.at[b], gsem[b]).wait()

        def s_start(i, b):
            pltpu.async_copy(rows_v.at[b], acc.at[dst_v.at[i]], ssem[b], add=True)

        def s_wait(b):
            pltpu.make_async_copy(rows_v.at[b], acc.at[dst_v.at[0]], ssem[b]).wait()

        @pl.when(c == 0)
        def _():
            pltpu.sync_copy(src0_hbm.at[s], src_v.at[pl.ds(0, NCH0)])
            pltpu.sync_copy(dst0_hbm.at[s], dst_v.at[pl.ds(0, NCH0)])

        @pl.when(c != 0)
        def _():
            pltpu.sync_copy(src1_hbm.at[s], src_v.at[pl.ds(0, NCH1)])
            pltpu.sync_copy(dst1_hbm.at[s], dst_v.at[pl.ds(0, NCH1)])

        _zero_fill(rows_v.at[0])
        for j in range(ROWS_PER_TILE // K):
            pltpu.sync_copy(rows_v.at[0], acc.at[pl.ds(r0 + j * K, K)])
        plsc.subcore_barrier()

        g_start(0, 0)

        def pair(j, carry):
            for b in (0, 1):
                i = 2 * j + b

                @pl.when(i >= 1)
                def _():
                    s_wait(1 - b)

                @pl.when(i + 1 < ncw)
                def _():
                    g_start(i + 1, 1 - b)

                g_wait(b)
                s_start(i, b)
            return carry

        lax.fori_loop(0, ncw // 2, pair, 0)
        s_wait(1)
        plsc.subcore_barrier()
        pltpu.sync_copy(acc.at[rows], out_hbm.at[c, rows])

    return edge_pass


_edge_pass_128 = _make_edge_pass(128)


@functools.partial(
    pl.kernel,
    out_type=jax.ShapeDtypeStruct((NC, NPAD, 128), jnp.float32),
    mesh=_MESH,
    scratch_types=[
        pltpu.VMEM((NCH0, K), jnp.int32),
        pltpu.VMEM((K, 128), jnp.float32),
        pltpu.VMEM_SHARED((NPAD, 128), jnp.float32),
        pltpu.SemaphoreType.DMA,
    ],
)
def _deg_pass(dst0_hbm, dst1_hbm, out_hbm, dst_v, ones_v, acc, sem):
    c = lax.axis_index("c")
    s = lax.axis_index("s")
    r0 = s * ROWS_PER_TILE
    rows = pl.ds(r0, ROWS_PER_TILE)
    ncw = jnp.where(c == 0, NCH0, NCH1)

    @pl.when(c == 0)
    def _():
        pltpu.sync_copy(dst0_hbm.at[s], dst_v.at[pl.ds(0, NCH0)])

    @pl.when(c != 0)
    def _():
        pltpu.sync_copy(dst1_hbm.at[s], dst_v.at[pl.ds(0, NCH1)])

    _zero_fill(ones_v)
    for j in range(ROWS_PER_TILE // K):
        pltpu.sync_copy(ones_v, acc.at[pl.ds(r0 + j * K, K)])

    def fill1(i, carry):
        ones_v[i // 8, pl.ds((i % 8) * 16, 16)] = jnp.ones((16,), jnp.float32)
        return carry

    lax.fori_loop(0, K * 8, fill1, 0)
    plsc.subcore_barrier()

    def chunk(i, carry):
        pltpu.async_copy(ones_v, acc.at[dst_v.at[i]], sem, add=True)
        return carry

    lax.fori_loop(0, ncw, chunk, 0)

    def drain(i, carry):
        pltpu.make_async_copy(ones_v, acc.at[dst_v.at[0]], sem).wait()
        return carry

    lax.fori_loop(0, ncw, drain, 0)
    plsc.subcore_barrier()
    pltpu.sync_copy(acc.at[rows], out_hbm.at[c, rows])


def _mm1(xp, w1, p0, p1):
    def body(x_ref, w_ref, p0_ref, p1_ref, h_ref, dinv_ref):
        deg = 1.0 + p0_ref[:, 0:1] + p1_ref[:, 0:1]
        dv = lax.rsqrt(deg)
        h = jnp.dot(x_ref[...], w_ref[...], preferred_element_type=jnp.float32)
        h_ref[...] = h * dv
        dinv_ref[...] = dv

    return pl.pallas_call(
        body,
        grid=(NPAD // BM,),
        in_specs=[
            pl.BlockSpec((BM, 384), lambda i: (i, 0)),
            pl.BlockSpec((384, 128), lambda i: (0, 0)),
            pl.BlockSpec((BM, 128), lambda i: (i, 0)),
            pl.BlockSpec((BM, 128), lambda i: (i, 0)),
        ],
        out_specs=[
            pl.BlockSpec((BM, 128), lambda i: (i, 0)),
            pl.BlockSpec((BM, 1), lambda i: (i, 0)),
        ],
        out_shape=[
            jax.ShapeDtypeStruct((NPAD, 128), jnp.float32),
            jax.ShapeDtypeStruct((NPAD, 1), jnp.float32),
        ],
    )(xp, w1, p0, p1)


def _epi_mm(p0, p1, h, dinv, b, w):
    def body(p0_ref, p1_ref, h_ref, dinv_ref, b_ref, w_ref, o_ref):
        dv = dinv_ref[...]
        act = jnp.maximum(
            dv * (p0_ref[...] + p1_ref[...] + h_ref[...]) + b_ref[...], 0.0)
        o_ref[...] = dv * jnp.dot(act, w_ref[...], preferred_element_type=jnp.float32)

    return pl.pallas_call(
        body,
        grid=(NPAD // BM,),
        in_specs=[
            pl.BlockSpec((BM, 128), lambda i: (i, 0)),
            pl.BlockSpec((BM, 128), lambda i: (i, 0)),
            pl.BlockSpec((BM, 128), lambda i: (i, 0)),
            pl.BlockSpec((BM, 1), lambda i: (i, 0)),
            pl.BlockSpec((1, 128), lambda i: (0, 0)),
            pl.BlockSpec((128, 128), lambda i: (0, 0)),
        ],
        out_specs=pl.BlockSpec((BM, 128), lambda i: (i, 0)),
        out_shape=jax.ShapeDtypeStruct((NPAD, 128), jnp.float32),
    )(p0, p1, h, dinv, b, w)


def _epi_final(p0, p1, h, dinv, b):
    def body(p0_ref, p1_ref, h_ref, dinv_ref, b_ref, o_ref):
        o_ref[...] = (dinv_ref[...] * (p0_ref[...] + p1_ref[...] + h_ref[...])
                      + b_ref[...])

    return pl.pallas_call(
        body,
        grid=(NPAD // BM,),
        in_specs=[
            pl.BlockSpec((BM, 128), lambda i: (i, 0)),
            pl.BlockSpec((BM, 128), lambda i: (i, 0)),
            pl.BlockSpec((BM, 128), lambda i: (i, 0)),
            pl.BlockSpec((BM, 1), lambda i: (i, 0)),
            pl.BlockSpec((1, 128), lambda i: (0, 0)),
        ],
        out_specs=pl.BlockSpec((BM, 128), lambda i: (i, 0)),
        out_shape=jax.ShapeDtypeStruct((NPAD, 128), jnp.float32),
    )(p0, p1, h, dinv, b)


def kernel(x, edge_index, W1, b1, W2, b2, W3, b3, W4, b4, W5, b5, W6, b6):
    xp = jnp.pad(x, ((0, NPAD - N), (0, 0)))
    srcf = jnp.concatenate([edge_index[0], jnp.zeros((EPAD - E,), jnp.int32)])
    pad_dst = N + jnp.arange(EPAD - E, dtype=jnp.int32) % (NPAD - N)
    dstf = jnp.concatenate([edge_index[1], pad_dst])
    cut = NS * NCH0 * K
    src0 = srcf[:cut].reshape(NS, NCH0, K)
    src1 = srcf[cut:].reshape(NS, NCH1, K)
    dst0 = dstf[:cut].reshape(NS, NCH0, K)
    dst1 = dstf[cut:].reshape(NS, NCH1, K)
    w1p = jnp.pad(W1, ((0, 0), (0, 16)))
    wp = [jnp.pad(w, ((0, 16), (0, 128 - w.shape[1]))) for w in (W2, W3, W4, W5, W6)]
    bp = [jnp.pad(b, (0, 128 - b.shape[0])).reshape(1, 128) for b in (b1, b2, b3, b4, b5, b6)]

    degp = _deg_pass(dst0, dst1)
    h, dinv = _mm1(xp, w1p, degp[0], degp[1])

    for bb, ww in zip(bp[:5], wp):
        p = _edge_pass_128(h, src0, dst0, src1, dst1)
        h = _epi_mm(p[0], p[1], h, dinv, bb, ww)

    p = _edge_pass_128(h, src0, dst0, src1, dst1)
    out = _epi_final(p[0], p[1], h, dinv, bp[5])
    return out[:N, :2]
